# Initial kernel scaffold; baseline (speedup 1.0000x reference)
import jax
import jax.numpy as jnp
from jax.experimental import pallas as pl


def kernel(hidden_states, gate_w, expert_gate_up_w, expert_down_w, shared_gate_up_w, shared_down_w, shared_expert_gate_w):
    raise NotImplementedError("write your pallas kernel here")


# speedup vs baseline: 2.2759x; 2.2759x over previous
import jax
import jax.numpy as jnp
from jax.experimental import pallas as pl
from jax.experimental.pallas import tpu as pltpu

HID = 1024
NE = 16
FF = 512
SFF = 2048
T = 512

_PREC = jax.lax.Precision.DEFAULT


def _dot_t(a, b, precision=_PREC):
    return jax.lax.dot_general(
        a, b, (((1,), (1,)), ((), ())),
        preferred_element_type=jnp.float32,
        precision=precision)


def _moe_body(x_ref, gate_w_ref, segw_ref, egu_ref, edw_ref, sg_ref, su_ref,
              sdw_ref, out_ref, combine_ref, sharedw_ref):
    e = pl.program_id(0)
    x = x_ref[...]

    @pl.when(e == 0)
    def _init():
        logits = _dot_t(x, gate_w_ref[...])
        idx = jax.lax.broadcasted_iota(jnp.int32, (T, NE), 1)
        m1 = jnp.max(logits, axis=1, keepdims=True)
        i1 = jnp.min(jnp.where(logits == m1, idx, NE), axis=1, keepdims=True)
        masked = jnp.where(idx == i1, -jnp.inf, logits)
        m2 = jnp.max(masked, axis=1, keepdims=True)
        i2 = jnp.min(jnp.where(masked == m2, idx, NE), axis=1, keepdims=True)
        w1 = 1.0 / (1.0 + jnp.exp(m2 - m1))
        w2 = 1.0 - w1
        combine_ref[...] = (jnp.where(idx == i1, w1, 0.0)
                            + jnp.where(idx == i2, w2, 0.0))
        sw = _dot_t(x, segw_ref[...])
        sharedw_ref[...] = jax.nn.sigmoid(sw)
        out_ref[...] = jnp.zeros_like(out_ref)

    gu = _dot_t(x, egu_ref[0])
    gate, up = gu[:, :FF], gu[:, FF:]
    act = gate * jax.nn.sigmoid(gate) * up
    eout = _dot_t(act, edw_ref[0])
    onehot = (jax.lax.broadcasted_iota(jnp.int32, (1, NE), 1) == e
              ).astype(jnp.float32)
    c_col = jnp.sum(combine_ref[...] * onehot, axis=1, keepdims=True)

    g = _dot_t(x, sg_ref[...])
    u = _dot_t(x, su_ref[...])
    a = g * jax.nn.sigmoid(g) * u
    sout = _dot_t(a, sdw_ref[...])
    out_ref[...] += eout * c_col + sout * sharedw_ref[...]


def kernel(hidden_states, gate_w, expert_gate_up_w, expert_down_w,
           shared_gate_up_w, shared_down_w, shared_expert_gate_w):
    orig_shape = hidden_states.shape
    x = hidden_states.reshape(T, HID)
    sc = SFF // NE

    out = pl.pallas_call(
        _moe_body,
        grid=(NE,),
        in_specs=[
            pl.BlockSpec((T, HID), lambda e: (0, 0)),
            pl.BlockSpec((NE, HID), lambda e: (0, 0)),
            pl.BlockSpec((1, HID), lambda e: (0, 0)),
            pl.BlockSpec((1, 2 * FF, HID), lambda e: (e, 0, 0)),
            pl.BlockSpec((1, HID, FF), lambda e: (e, 0, 0)),
            pl.BlockSpec((sc, HID), lambda e: (e, 0)),
            pl.BlockSpec((sc, HID), lambda e: (e + NE, 0)),
            pl.BlockSpec((HID, sc), lambda e: (0, e)),
        ],
        out_specs=pl.BlockSpec((T, HID), lambda e: (0, 0)),
        out_shape=jax.ShapeDtypeStruct((T, HID), jnp.float32),
        scratch_shapes=[
            pltpu.VMEM((T, NE), jnp.float32),
            pltpu.VMEM((T, 1), jnp.float32),
        ],
        compiler_params=pltpu.CompilerParams(
            dimension_semantics=("arbitrary",)),
    )(x, gate_w, shared_expert_gate_w, expert_gate_up_w, expert_down_w,
      shared_gate_up_w, shared_gate_up_w, shared_down_w)
    return out.reshape(orig_shape)
